# Initial kernel scaffold; baseline (speedup 1.0000x reference)
import jax
import jax.numpy as jnp
from jax.experimental import pallas as pl


def kernel(users_emb, items_emb, edge_index, edge_weight):
    raise NotImplementedError("write your pallas kernel here")


# speedup vs baseline: 2.8615x; 2.8615x over previous
import jax
import jax.numpy as jnp
from jax import lax
from jax.experimental import pallas as pl
from jax.experimental.pallas import tpu as pltpu
from jax.experimental.pallas import tpu_sc as plsc

N_USERS = 6000
N_ITEMS = 4000
N = N_USERS + N_ITEMS
E = 320000
D = 128
HALF = D // 2

NC = 2
NS = 16
NP = 10240
CHUNK = 80
SCH = 10
NSC = E // CHUNK // NS // SCH
RPT = NP // NS


def _sc_body(tabs, src, dst, wgt, out, spa, spb, srcsc, dstsc, wsc, msg,
             suml, sem):
    c = lax.axis_index("c")
    s = lax.axis_index("s")
    row0 = s * RPT

    pltpu.sync_copy(tabs.at[c].at[pl.ds(row0, RPT)], suml)

    gdn = lax.GatherDimensionNumbers(
        offset_dims=(), collapsed_slice_dims=(0,), start_index_map=(0,))

    def zero_msg(i, carry):
        for k in range(4):
            msg[i, pl.ds(k * 16, 16)] = jnp.zeros((16,), jnp.float32)
        return carry

    def propagate(gather_from, tgt):
        lax.fori_loop(0, CHUNK, zero_msg, 0)
        for k in range(RPT // CHUNK):
            pltpu.sync_copy(msg, tgt.at[pl.ds(row0 + k * CHUNK, CHUNK)])
        plsc.subcore_barrier()

        def sc_body(q, carry):
            pltpu.sync_copy(src.at[s].at[q], srcsc)
            pltpu.sync_copy(dst.at[s].at[q], dstsc)
            pltpu.sync_copy(wgt.at[s].at[q], wsc)

            def one_chunk(r, carry2):
                pltpu.async_copy(gather_from.at[srcsc.at[r]], msg, sem).wait()

                def weight_rows(g, carry3):
                    wv = wsc[r, pl.ds(g * 16, 16)]
                    for j in range(16):
                        row = g * 16 + j
                        wspl = lax.gather(
                            wv, jnp.full((16, 1), j, jnp.int32), gdn,
                            slice_sizes=(1,),
                            mode=lax.GatherScatterMode.PROMISE_IN_BOUNDS)
                        for k in range(4):
                            sl = pl.ds(k * 16, 16)
                            msg[row, sl] = msg[row, sl] * wspl
                    return carry3

                lax.fori_loop(0, CHUNK // 16, weight_rows, 0)
                pltpu.sync_copy(msg, tgt.at[dstsc.at[r]], add=True)
                return carry2

            lax.fori_loop(0, SCH, one_chunk, 0)
            return carry

        lax.fori_loop(0, NSC, sc_body, 0)
        plsc.subcore_barrier()

        for k in range(RPT // CHUNK):
            pltpu.sync_copy(tgt.at[pl.ds(row0 + k * CHUNK, CHUNK)], msg)

            def add_body(i, carry):
                for q in range(4):
                    sl = pl.ds(q * 16, 16)
                    suml[k * CHUNK + i, sl] = (
                        suml[k * CHUNK + i, sl] + msg[i, sl])
                return carry

            lax.fori_loop(0, CHUNK, add_body, 0)

    propagate(tabs.at[c], spa)
    propagate(spa, spb)
    propagate(spb, spa)

    def scale_body(i, carry):
        for q in range(4):
            sl = pl.ds(q * 16, 16)
            suml[i, sl] = suml[i, sl] * 0.25
        return carry

    lax.fori_loop(0, RPT, scale_body, 0)
    pltpu.sync_copy(suml, out.at[c].at[pl.ds(row0, RPT)])


@jax.jit
def _light_gcn(tabs, src4d, dst4d, w4d):
    mesh = plsc.VectorSubcoreMesh(
        core_axis_name="c", subcore_axis_name="s", num_cores=NC,
        num_subcores=NS)
    fn = pl.kernel(
        _sc_body,
        out_type=jax.ShapeDtypeStruct((NC, NP, HALF), jnp.float32),
        mesh=mesh,
        scratch_types=[
            pltpu.VMEM_SHARED((NP, HALF), jnp.float32),
            pltpu.VMEM_SHARED((NP, HALF), jnp.float32),
            pltpu.VMEM((SCH, CHUNK), jnp.int32),
            pltpu.VMEM((SCH, CHUNK), jnp.int32),
            pltpu.VMEM((SCH, CHUNK), jnp.float32),
            pltpu.VMEM((CHUNK, HALF), jnp.float32),
            pltpu.VMEM((RPT, HALF), jnp.float32),
            pltpu.SemaphoreType.DMA,
        ],
        compiler_params=pltpu.CompilerParams(use_tc_tiling_on_sc=False),
    )
    return fn(tabs, src4d, dst4d, w4d)


def kernel(users_emb, items_emb, edge_index, edge_weight):
    all0 = jnp.concatenate([users_emb, items_emb], axis=0)
    all0 = jnp.pad(all0, ((0, NP - N), (0, 0)))
    tabs = jnp.stack([all0[:, :HALF], all0[:, HALF:]])
    src4d = edge_index[0].reshape(NS, NSC, SCH, CHUNK)
    dst4d = edge_index[1].reshape(NS, NSC, SCH, CHUNK)
    w4d = edge_weight.reshape(NS, NSC, SCH, CHUNK)
    out = _light_gcn(tabs, src4d, dst4d, w4d)
    light = out.transpose(1, 0, 2).reshape(NP, D)[:N]
    return (light[:N_USERS], light[N_USERS:])
